# baseline (speedup 1.0000x reference)
import jax
import jax.numpy as jnp
from jax import lax
from jax.experimental import pallas as pl
from jax.experimental.pallas import tpu as pltpu
import jax.experimental.pallas.tpu_sc as plsc

N = 10000
D = 128
E = 320000
NC = 2
NS = 16
NW = NC * NS
EPT = E // NW
C = 128
NWIN = 80
EPTP = NWIN * C
NR = 10240
RPT = NR // NS
RB = 1000


def _mesh():
    return plsc.VectorSubcoreMesh(core_axis_name="c", subcore_axis_name="s")


def _deg_body(src3, dst3, zs2, degp, idx_s, idx_d, ones_v, acc_o, acc_i,
              sem_a, sem_b):
    cid = lax.axis_index("c")
    sid = lax.axis_index("s")
    wid = cid * NS + sid

    for j in range(8):
        ones_v[pl.ds(j * 16, 16)] = jnp.ones((16,), jnp.float32)

    @pl.when(sid == 0)
    def _():
        pltpu.sync_copy(zs2.at[0], acc_o)
        pltpu.sync_copy(zs2.at[1], acc_i)

    pltpu.sync_copy(src3.at[wid], idx_s)
    pltpu.sync_copy(dst3.at[wid], idx_d)
    plsc.subcore_barrier()

    def body(j, carry):
        a = pltpu.async_copy(ones_v, acc_o.at[idx_s.at[j]], sem_a, add=True)
        b = pltpu.async_copy(ones_v, acc_i.at[idx_d.at[j]], sem_b, add=True)
        a.wait()
        b.wait()
        return carry

    lax.fori_loop(0, NWIN, body, 0)
    plsc.subcore_barrier()

    off = sid * RPT
    pltpu.sync_copy(acc_o.at[pl.ds(off, RPT)], degp.at[cid, 0, pl.ds(off, RPT)])
    pltpu.sync_copy(acc_i.at[pl.ds(off, RPT)], degp.at[cid, 1, pl.ds(off, RPT)])


def _deg_call(src3, dst3, zs2):
    f = pl.kernel(
        _deg_body,
        out_type=jax.ShapeDtypeStruct((NC, 2, NR), jnp.float32),
        mesh=_mesh(),
        scratch_types=[
            pltpu.VMEM((NWIN, C), jnp.int32),
            pltpu.VMEM((NWIN, C), jnp.int32),
            pltpu.VMEM((C,), jnp.float32),
            pltpu.MemorySpace.VMEM_SHARED((NR,), jnp.float32),
            pltpu.MemorySpace.VMEM_SHARED((NR,), jnp.float32),
            pltpu.SemaphoreType.DMA,
            pltpu.SemaphoreType.DMA,
        ],
    )
    return f(src3, dst3, zs2)


ICH = 8
NCHK = NWIN // ICH


def _agg_body(m_hbm, src3, dst3, zrows, part, ics, icd, rows, acc,
              sem_g0, sem_g1, sem_s0, sem_s1):
    cid = lax.axis_index("c")
    sid = lax.axis_index("s")
    wid = cid * NS + sid

    pltpu.sync_copy(zrows.at[sid], acc.at[pl.ds(sid * RPT, RPT)])
    plsc.subcore_barrier()

    sem_s = (sem_s0, sem_s1)
    sem_g = (sem_g0, sem_g1)

    H = C // 2

    def _gather(k, b):
        ga = pltpu.async_copy(m_hbm.at[ics.at[k, pl.ds(0, H)]],
                              rows.at[b, pl.ds(0, H)], sem_g[b])
        gb = pltpu.async_copy(m_hbm.at[ics.at[k, pl.ds(H, H)]],
                              rows.at[b, pl.ds(H, H)], sem_g[b])
        return (ga, gb)

    def chunk(cix, carry):
        base = cix * ICH
        @pl.when(cix > 0)
        def _():
            pltpu.make_async_copy(rows.at[0], acc.at[icd.at[0]],
                                  sem_s0).wait()
            pltpu.make_async_copy(rows.at[1], acc.at[icd.at[1]],
                                  sem_s1).wait()

        pltpu.sync_copy(src3.at[wid, pl.ds(base, ICH)], ics)
        pltpu.sync_copy(dst3.at[wid, pl.ds(base, ICH)], icd)

        g = [_gather(0, 0), None]
        s = [None, None]
        for k in range(ICH):
            b = k % 2
            nb = 1 - b
            if k + 1 < ICH:
                if s[nb] is not None:
                    s[nb].wait()
                g[nb] = _gather(k + 1, nb)
            g[b][0].wait()
            g[b][1].wait()
            s[b] = pltpu.async_copy(rows.at[b], acc.at[icd.at[k]],
                                    sem_s[b], add=True)
        return carry

    lax.fori_loop(0, NCHK, chunk, 0)
    pltpu.make_async_copy(rows.at[0], acc.at[icd.at[0]], sem_s0).wait()
    pltpu.make_async_copy(rows.at[1], acc.at[icd.at[1]], sem_s1).wait()
    plsc.subcore_barrier()

    pltpu.sync_copy(acc.at[pl.ds(sid * RPT, RPT)],
                    part.at[cid, pl.ds(sid * RPT, RPT)])


def _agg_call(m, src3, dst3, zrows):
    f = pl.kernel(
        _agg_body,
        out_type=jax.ShapeDtypeStruct((NC, NR, D), jnp.float32),
        mesh=_mesh(),
        scratch_types=[
            pltpu.VMEM((ICH, C), jnp.int32),
            pltpu.VMEM((ICH, C), jnp.int32),
            pltpu.VMEM((2, C, D), jnp.float32),
            pltpu.MemorySpace.VMEM_SHARED((NR, D), jnp.float32),
            pltpu.SemaphoreType.DMA,
            pltpu.SemaphoreType.DMA,
            pltpu.SemaphoreType.DMA,
            pltpu.SemaphoreType.DMA,
        ],
    )
    return f(m, src3, dst3, zrows)


def _stage_a_body(no_ref, h_ref, w_ref, o_ref):
    o_ref[...] = jnp.dot(h_ref[...] * no_ref[...], w_ref[...],
                         preferred_element_type=jnp.float32)


def _stage_a(norm_out, h, W1):
    return pl.pallas_call(
        _stage_a_body,
        out_shape=jax.ShapeDtypeStruct((NR, D), jnp.float32),
        grid=(N // RB,),
        in_specs=[
            pl.BlockSpec((RB, 1), lambda i: (i, 0)),
            pl.BlockSpec((RB, D), lambda i: (i, 0)),
            pl.BlockSpec((D, D), lambda i: (0, 0)),
        ],
        out_specs=pl.BlockSpec((RB, D), lambda i: (i, 0)),
    )(norm_out, h, W1)


def _stage_b_body(p_ref, ni_ref, b1_ref, g_ref, be_ref, no_ref, w2_ref,
                  m2_ref, x_vmem, s_vmem):
    ph = pl.program_id(0)
    i = pl.program_id(1)

    @pl.when(ph == 0)
    def _():
        x = (p_ref[0] + p_ref[1]) * ni_ref[...] + b1_ref[...]
        x_vmem[i] = x
        s0 = jnp.sum(x, axis=0)[None, :]
        s1 = jnp.sum(x * x, axis=0)[None, :]

        @pl.when(i == 0)
        def _():
            s_vmem[0:1, :] = s0
            s_vmem[1:2, :] = s1

        @pl.when(i > 0)
        def _():
            s_vmem[0:1, :] += s0
            s_vmem[1:2, :] += s1

    @pl.when(ph == 1)
    def _():
        inv_n = jnp.float32(1.0 / N)
        mean = s_vmem[0:1, :] * inv_n
        ex2 = s_vmem[1:2, :] * inv_n
        var = ex2 - mean * mean
        xb = x_vmem[i]
        xn = (xb - mean) * lax.rsqrt(var + 1e-5) * g_ref[...] + be_ref[...]
        r = jnp.maximum(xn, 0.0) * no_ref[...]
        m2_ref[...] = jnp.dot(r, w2_ref[...],
                              preferred_element_type=jnp.float32)


def _stage_b(p1, norm_in, b1, gamma, beta, norm_out, W2):
    return pl.pallas_call(
        _stage_b_body,
        out_shape=jax.ShapeDtypeStruct((NR, D), jnp.float32),
        grid=(2, N // RB),
        in_specs=[
            pl.BlockSpec((2, RB, D), lambda p, i: (0, i * (1 - p), 0)),
            pl.BlockSpec((RB, 1), lambda p, i: (i, 0)),
            pl.BlockSpec((1, D), lambda p, i: (0, 0)),
            pl.BlockSpec((1, D), lambda p, i: (0, 0)),
            pl.BlockSpec((1, D), lambda p, i: (0, 0)),
            pl.BlockSpec((RB, 1), lambda p, i: (i, 0)),
            pl.BlockSpec((D, D), lambda p, i: (0, 0)),
        ],
        out_specs=pl.BlockSpec((RB, D), lambda p, i: (i * p, 0)),
        scratch_shapes=[
            pltpu.VMEM((N // RB, RB, D), jnp.float32),
            pltpu.VMEM((8, D), jnp.float32),
        ],
    )(p1, norm_in, b1, gamma, beta, norm_out, W2)


def _stage_c_body(q_ref, ni_ref, b2_ref, o_ref):
    o_ref[...] = (q_ref[0] + q_ref[1]) * ni_ref[...] + b2_ref[...]


def _stage_c(p2, norm_in, b2):
    return pl.pallas_call(
        _stage_c_body,
        out_shape=jax.ShapeDtypeStruct((N, D), jnp.float32),
        grid=(N // RB,),
        in_specs=[
            pl.BlockSpec((2, RB, D), lambda i: (0, i, 0)),
            pl.BlockSpec((RB, 1), lambda i: (i, 0)),
            pl.BlockSpec((1, D), lambda i: (0, 0)),
        ],
        out_specs=pl.BlockSpec((RB, D), lambda i: (i, 0)),
    )(p2, norm_in, b2)


def kernel(h, edge_index, W1, b1, gamma, beta, W2, b2):
    npad = EPTP - EPT
    padi = (N + jnp.arange(npad, dtype=jnp.int32) % (NR - N))[None, :]
    src2 = edge_index[0].reshape(NW, EPT)
    dst2 = edge_index[1].reshape(NW, EPT)
    pads = jnp.broadcast_to(padi, (NW, npad))
    src3 = jnp.concatenate([src2, pads], axis=1).reshape(NW, NWIN, C)
    dst3 = jnp.concatenate([dst2, pads], axis=1).reshape(NW, NWIN, C)

    zs2 = jnp.zeros((2, NR), jnp.float32)
    zrows = jnp.zeros((NS, RPT, D), jnp.float32)

    degp = _deg_call(src3, dst3, zs2)
    deg_out = degp[0, 0, :N] + degp[1, 0, :N]
    deg_in = degp[0, 1, :N] + degp[1, 1, :N]
    norm_out = jnp.where(deg_out > 0,
                         lax.rsqrt(jnp.maximum(deg_out, 1.0)),
                         0.0).reshape(N, 1)
    norm_in = jnp.where(deg_in > 0,
                        lax.rsqrt(jnp.maximum(deg_in, 1.0)),
                        0.0).reshape(N, 1)

    m1 = _stage_a(norm_out, h, W1)
    p1 = _agg_call(m1, src3, dst3, zrows)
    m2 = _stage_b(p1, norm_in, b1.reshape(1, D), gamma.reshape(1, D),
                  beta.reshape(1, D), norm_out, W2)
    p2 = _agg_call(m2, src3, dst3, zrows)
    return _stage_c(p2, norm_in, b2.reshape(1, D))


# speedup vs baseline: 1.0701x; 1.0701x over previous
import jax
import jax.numpy as jnp
from jax import lax
from jax.experimental import pallas as pl
from jax.experimental.pallas import tpu as pltpu
import jax.experimental.pallas.tpu_sc as plsc

N = 10000
D = 128
E = 320000
NC = 2
NS = 16
NW = NC * NS
EPT = E // NW
C = 128
NWIN = 80
EPTP = NWIN * C
NR = 10240
RPT = NR // NS
RB = 1000


def _mesh():
    return plsc.VectorSubcoreMesh(core_axis_name="c", subcore_axis_name="s")


def _deg_body(src3, dst3, zs2, degp, idx_s, idx_d, ones_v, acc_o, acc_i,
              sem_a, sem_b):
    cid = lax.axis_index("c")
    sid = lax.axis_index("s")
    wid = cid * NS + sid

    for j in range(8):
        ones_v[pl.ds(j * 16, 16)] = jnp.ones((16,), jnp.float32)

    @pl.when(sid == 0)
    def _():
        pltpu.sync_copy(zs2.at[0], acc_o)
        pltpu.sync_copy(zs2.at[1], acc_i)

    pltpu.sync_copy(src3.at[wid], idx_s)
    pltpu.sync_copy(dst3.at[wid], idx_d)
    plsc.subcore_barrier()

    def body(j, carry):
        a = pltpu.async_copy(ones_v, acc_o.at[idx_s.at[j]], sem_a, add=True)
        b = pltpu.async_copy(ones_v, acc_i.at[idx_d.at[j]], sem_b, add=True)
        a.wait()
        b.wait()
        return carry

    lax.fori_loop(0, NWIN, body, 0)
    plsc.subcore_barrier()

    off = sid * RPT
    pltpu.sync_copy(acc_o.at[pl.ds(off, RPT)], degp.at[cid, 0, pl.ds(off, RPT)])
    pltpu.sync_copy(acc_i.at[pl.ds(off, RPT)], degp.at[cid, 1, pl.ds(off, RPT)])


def _deg_call(src3, dst3, zs2):
    f = pl.kernel(
        _deg_body,
        out_type=jax.ShapeDtypeStruct((NC, 2, NR), jnp.float32),
        mesh=_mesh(),
        scratch_types=[
            pltpu.VMEM((NWIN, C), jnp.int32),
            pltpu.VMEM((NWIN, C), jnp.int32),
            pltpu.VMEM((C,), jnp.float32),
            pltpu.MemorySpace.VMEM_SHARED((NR,), jnp.float32),
            pltpu.MemorySpace.VMEM_SHARED((NR,), jnp.float32),
            pltpu.SemaphoreType.DMA,
            pltpu.SemaphoreType.DMA,
        ],
    )
    return f(src3, dst3, zs2)


ICH = 8
NCHK = NWIN // ICH


def _agg_body(m_hbm, src3, dst3, zrows, part, ics, icd, rows, acc,
              sem_g0, sem_g1, sem_s0, sem_s1, sem_i):
    cid = lax.axis_index("c")
    sid = lax.axis_index("s")
    wid = cid * NS + sid

    pltpu.sync_copy(zrows.at[sid], acc.at[pl.ds(sid * RPT, RPT)])

    sem_s = (sem_s0, sem_s1)
    sem_g = (sem_g0, sem_g1)

    Q = C // 4

    def _gather(slot, k, b):
        return [pltpu.async_copy(m_hbm.at[ics.at[slot, k, pl.ds(q * Q, Q)]],
                                 rows.at[b, pl.ds(q * Q, Q)], sem_g[b])
                for q in range(4)]

    pltpu.async_copy(src3.at[wid, pl.ds(0, ICH)], ics.at[0], sem_i)
    pltpu.async_copy(dst3.at[wid, pl.ds(0, ICH)], icd.at[0], sem_i)
    plsc.subcore_barrier()

    def chunk(cix, carry):
        slot = lax.rem(cix, 2)
        nslot = 1 - slot
        pltpu.make_async_copy(src3.at[wid, pl.ds(0, ICH)], ics.at[slot],
                              sem_i).wait()
        pltpu.make_async_copy(dst3.at[wid, pl.ds(0, ICH)], icd.at[slot],
                              sem_i).wait()

        @pl.when(cix > 0)
        def _():
            pltpu.make_async_copy(rows.at[0], acc.at[icd.at[0, 0]],
                                  sem_s0).wait()
            pltpu.make_async_copy(rows.at[1], acc.at[icd.at[1, 0]],
                                  sem_s1).wait()

        @pl.when(cix + 1 < NCHK)
        def _():
            nbase = (cix + 1) * ICH
            pltpu.async_copy(src3.at[wid, pl.ds(nbase, ICH)], ics.at[nslot],
                             sem_i)
            pltpu.async_copy(dst3.at[wid, pl.ds(nbase, ICH)], icd.at[nslot],
                             sem_i)

        g = [_gather(slot, 0, 0), None]
        s = [None, None]
        for k in range(ICH):
            b = k % 2
            nb = 1 - b
            if k + 1 < ICH:
                if s[nb] is not None:
                    s[nb].wait()
                g[nb] = _gather(slot, k + 1, nb)
            for gd in g[b]:
                gd.wait()
            s[b] = pltpu.async_copy(rows.at[b], acc.at[icd.at[slot, k]],
                                    sem_s[b], add=True)
        return carry

    lax.fori_loop(0, NCHK, chunk, 0)
    pltpu.make_async_copy(rows.at[0], acc.at[icd.at[0, 0]], sem_s0).wait()
    pltpu.make_async_copy(rows.at[1], acc.at[icd.at[1, 0]], sem_s1).wait()
    plsc.subcore_barrier()

    pltpu.sync_copy(acc.at[pl.ds(sid * RPT, RPT)],
                    part.at[cid, pl.ds(sid * RPT, RPT)])


def _agg_call(m, src3, dst3, zrows):
    f = pl.kernel(
        _agg_body,
        out_type=jax.ShapeDtypeStruct((NC, NR, D), jnp.float32),
        mesh=_mesh(),
        scratch_types=[
            pltpu.VMEM((2, ICH, C), jnp.int32),
            pltpu.VMEM((2, ICH, C), jnp.int32),
            pltpu.VMEM((2, C, D), jnp.float32),
            pltpu.MemorySpace.VMEM_SHARED((NR, D), jnp.float32),
            pltpu.SemaphoreType.DMA,
            pltpu.SemaphoreType.DMA,
            pltpu.SemaphoreType.DMA,
            pltpu.SemaphoreType.DMA,
            pltpu.SemaphoreType.DMA,
        ],
    )
    return f(m, src3, dst3, zrows)


def _stage_a_body(no_ref, h_ref, w_ref, o_ref):
    o_ref[...] = jnp.dot(h_ref[...] * no_ref[...], w_ref[...],
                         preferred_element_type=jnp.float32)


def _stage_a(norm_out, h, W1):
    return pl.pallas_call(
        _stage_a_body,
        out_shape=jax.ShapeDtypeStruct((NR, D), jnp.float32),
        grid=(N // RB,),
        in_specs=[
            pl.BlockSpec((RB, 1), lambda i: (i, 0)),
            pl.BlockSpec((RB, D), lambda i: (i, 0)),
            pl.BlockSpec((D, D), lambda i: (0, 0)),
        ],
        out_specs=pl.BlockSpec((RB, D), lambda i: (i, 0)),
    )(norm_out, h, W1)


def _stage_b_body(p_ref, ni_ref, b1_ref, g_ref, be_ref, no_ref, w2_ref,
                  m2_ref, x_vmem, s_vmem):
    ph = pl.program_id(0)
    i = pl.program_id(1)

    @pl.when(ph == 0)
    def _():
        x = (p_ref[0] + p_ref[1]) * ni_ref[...] + b1_ref[...]
        x_vmem[i] = x
        s0 = jnp.sum(x, axis=0)[None, :]
        s1 = jnp.sum(x * x, axis=0)[None, :]

        @pl.when(i == 0)
        def _():
            s_vmem[0:1, :] = s0
            s_vmem[1:2, :] = s1

        @pl.when(i > 0)
        def _():
            s_vmem[0:1, :] += s0
            s_vmem[1:2, :] += s1

    @pl.when(ph == 1)
    def _():
        inv_n = jnp.float32(1.0 / N)
        mean = s_vmem[0:1, :] * inv_n
        ex2 = s_vmem[1:2, :] * inv_n
        var = ex2 - mean * mean
        xb = x_vmem[i]
        xn = (xb - mean) * lax.rsqrt(var + 1e-5) * g_ref[...] + be_ref[...]
        r = jnp.maximum(xn, 0.0) * no_ref[...]
        m2_ref[...] = jnp.dot(r, w2_ref[...],
                              preferred_element_type=jnp.float32)


def _stage_b(p1, norm_in, b1, gamma, beta, norm_out, W2):
    return pl.pallas_call(
        _stage_b_body,
        out_shape=jax.ShapeDtypeStruct((NR, D), jnp.float32),
        grid=(2, N // RB),
        in_specs=[
            pl.BlockSpec((2, RB, D), lambda p, i: (0, i * (1 - p), 0)),
            pl.BlockSpec((RB, 1), lambda p, i: (i, 0)),
            pl.BlockSpec((1, D), lambda p, i: (0, 0)),
            pl.BlockSpec((1, D), lambda p, i: (0, 0)),
            pl.BlockSpec((1, D), lambda p, i: (0, 0)),
            pl.BlockSpec((RB, 1), lambda p, i: (i, 0)),
            pl.BlockSpec((D, D), lambda p, i: (0, 0)),
        ],
        out_specs=pl.BlockSpec((RB, D), lambda p, i: (i * p, 0)),
        scratch_shapes=[
            pltpu.VMEM((N // RB, RB, D), jnp.float32),
            pltpu.VMEM((8, D), jnp.float32),
        ],
    )(p1, norm_in, b1, gamma, beta, norm_out, W2)


def _stage_c_body(q_ref, ni_ref, b2_ref, o_ref):
    o_ref[...] = (q_ref[0] + q_ref[1]) * ni_ref[...] + b2_ref[...]


def _stage_c(p2, norm_in, b2):
    return pl.pallas_call(
        _stage_c_body,
        out_shape=jax.ShapeDtypeStruct((N, D), jnp.float32),
        grid=(N // RB,),
        in_specs=[
            pl.BlockSpec((2, RB, D), lambda i: (0, i, 0)),
            pl.BlockSpec((RB, 1), lambda i: (i, 0)),
            pl.BlockSpec((1, D), lambda i: (0, 0)),
        ],
        out_specs=pl.BlockSpec((RB, D), lambda i: (i, 0)),
    )(p2, norm_in, b2)


def kernel(h, edge_index, W1, b1, gamma, beta, W2, b2):
    npad = EPTP - EPT
    padi = (N + jnp.arange(npad, dtype=jnp.int32) % (NR - N))[None, :]
    src2 = edge_index[0].reshape(NW, EPT)
    dst2 = edge_index[1].reshape(NW, EPT)
    pads = jnp.broadcast_to(padi, (NW, npad))
    src3 = jnp.concatenate([src2, pads], axis=1).reshape(NW, NWIN, C)
    dst3 = jnp.concatenate([dst2, pads], axis=1).reshape(NW, NWIN, C)

    zs2 = jnp.zeros((2, NR), jnp.float32)
    zrows = jnp.zeros((NS, RPT, D), jnp.float32)

    degp = _deg_call(src3, dst3, zs2)
    deg_out = degp[0, 0, :N] + degp[1, 0, :N]
    deg_in = degp[0, 1, :N] + degp[1, 1, :N]
    norm_out = jnp.where(deg_out > 0,
                         lax.rsqrt(jnp.maximum(deg_out, 1.0)),
                         0.0).reshape(N, 1)
    norm_in = jnp.where(deg_in > 0,
                        lax.rsqrt(jnp.maximum(deg_in, 1.0)),
                        0.0).reshape(N, 1)

    m1 = _stage_a(norm_out, h, W1)
    p1 = _agg_call(m1, src3, dst3, zrows)
    m2 = _stage_b(p1, norm_in, b1.reshape(1, D), gamma.reshape(1, D),
                  beta.reshape(1, D), norm_out, W2)
    p2 = _agg_call(m2, src3, dst3, zrows)
    return _stage_c(p2, norm_in, b2.reshape(1, D))


# speedup vs baseline: 1.0825x; 1.0116x over previous
import jax
import jax.numpy as jnp
from jax import lax
from jax.experimental import pallas as pl
from jax.experimental.pallas import tpu as pltpu
import jax.experimental.pallas.tpu_sc as plsc

N = 10000
D = 128
E = 320000
NC = 2
NS = 16
NW = NC * NS
EPT = E // NW
C = 128
NWIN = 80
EPTP = NWIN * C
NR = 10240
RPT = NR // NS
RB = 1000


def _mesh():
    return plsc.VectorSubcoreMesh(core_axis_name="c", subcore_axis_name="s")


def _deg_body(src3, dst3, zs2, degp, idx_s, idx_d, ones_v, acc_o, acc_i,
              sem_a, sem_b):
    cid = lax.axis_index("c")
    sid = lax.axis_index("s")
    wid = cid * NS + sid

    for j in range(8):
        ones_v[pl.ds(j * 16, 16)] = jnp.ones((16,), jnp.float32)

    @pl.when(sid == 0)
    def _():
        pltpu.sync_copy(zs2.at[0], acc_o)
        pltpu.sync_copy(zs2.at[1], acc_i)

    pltpu.sync_copy(src3.at[wid], idx_s)
    pltpu.sync_copy(dst3.at[wid], idx_d)
    plsc.subcore_barrier()

    def body(j4, carry):
        ds = []
        for t in range(4):
            j = j4 * 4 + t
            ds.append(pltpu.async_copy(ones_v, acc_o.at[idx_s.at[j]],
                                       sem_a, add=True))
            ds.append(pltpu.async_copy(ones_v, acc_i.at[idx_d.at[j]],
                                       sem_b, add=True))
        for dsc in ds:
            dsc.wait()
        return carry

    lax.fori_loop(0, NWIN // 4, body, 0)
    plsc.subcore_barrier()

    off = sid * RPT
    pltpu.sync_copy(acc_o.at[pl.ds(off, RPT)], degp.at[cid, 0, pl.ds(off, RPT)])
    pltpu.sync_copy(acc_i.at[pl.ds(off, RPT)], degp.at[cid, 1, pl.ds(off, RPT)])


def _deg_call(src3, dst3, zs2):
    f = pl.kernel(
        _deg_body,
        out_type=jax.ShapeDtypeStruct((NC, 2, NR), jnp.float32),
        mesh=_mesh(),
        scratch_types=[
            pltpu.VMEM((NWIN, C), jnp.int32),
            pltpu.VMEM((NWIN, C), jnp.int32),
            pltpu.VMEM((C,), jnp.float32),
            pltpu.MemorySpace.VMEM_SHARED((NR,), jnp.float32),
            pltpu.MemorySpace.VMEM_SHARED((NR,), jnp.float32),
            pltpu.SemaphoreType.DMA,
            pltpu.SemaphoreType.DMA,
        ],
    )
    return f(src3, dst3, zs2)


ICH = 8
NCHK = NWIN // ICH


def _agg_body(m_hbm, src3, dst3, zrows, part, ics, icd, rows, acc,
              sem_g0, sem_g1, sem_s0, sem_s1, sem_i):
    cid = lax.axis_index("c")
    sid = lax.axis_index("s")
    wid = cid * NS + sid

    pltpu.sync_copy(zrows.at[sid], acc.at[pl.ds(sid * RPT, RPT)])

    sem_s = (sem_s0, sem_s1)
    sem_g = (sem_g0, sem_g1)

    Q = C // 4

    def _gather(slot, k, b):
        return [pltpu.async_copy(m_hbm.at[ics.at[slot, k, pl.ds(q * Q, Q)]],
                                 rows.at[b, pl.ds(q * Q, Q)], sem_g[b])
                for q in range(4)]

    pltpu.async_copy(src3.at[wid, pl.ds(0, ICH)], ics.at[0], sem_i)
    pltpu.async_copy(dst3.at[wid, pl.ds(0, ICH)], icd.at[0], sem_i)
    plsc.subcore_barrier()

    def chunk(cix, carry):
        slot = lax.rem(cix, 2)
        nslot = 1 - slot
        pltpu.make_async_copy(src3.at[wid, pl.ds(0, ICH)], ics.at[slot],
                              sem_i).wait()
        pltpu.make_async_copy(dst3.at[wid, pl.ds(0, ICH)], icd.at[slot],
                              sem_i).wait()

        @pl.when(cix > 0)
        def _():
            pltpu.make_async_copy(rows.at[0], acc.at[icd.at[0, 0]],
                                  sem_s0).wait()
            pltpu.make_async_copy(rows.at[1], acc.at[icd.at[1, 0]],
                                  sem_s1).wait()

        @pl.when(cix + 1 < NCHK)
        def _():
            nbase = (cix + 1) * ICH
            pltpu.async_copy(src3.at[wid, pl.ds(nbase, ICH)], ics.at[nslot],
                             sem_i)
            pltpu.async_copy(dst3.at[wid, pl.ds(nbase, ICH)], icd.at[nslot],
                             sem_i)

        g = [_gather(slot, 0, 0), None]
        s = [None, None]
        for k in range(ICH):
            b = k % 2
            nb = 1 - b
            if k + 1 < ICH:
                if s[nb] is not None:
                    s[nb].wait()
                g[nb] = _gather(slot, k + 1, nb)
            for gd in g[b]:
                gd.wait()
            s[b] = pltpu.async_copy(rows.at[b], acc.at[icd.at[slot, k]],
                                    sem_s[b], add=True)
        return carry

    lax.fori_loop(0, NCHK, chunk, 0)
    pltpu.make_async_copy(rows.at[0], acc.at[icd.at[0, 0]], sem_s0).wait()
    pltpu.make_async_copy(rows.at[1], acc.at[icd.at[1, 0]], sem_s1).wait()
    plsc.subcore_barrier()

    pltpu.sync_copy(acc.at[pl.ds(sid * RPT, RPT)],
                    part.at[cid, pl.ds(sid * RPT, RPT)])


def _agg_call(m, src3, dst3, zrows):
    f = pl.kernel(
        _agg_body,
        out_type=jax.ShapeDtypeStruct((NC, NR, D), jnp.float32),
        mesh=_mesh(),
        scratch_types=[
            pltpu.VMEM((2, ICH, C), jnp.int32),
            pltpu.VMEM((2, ICH, C), jnp.int32),
            pltpu.VMEM((2, C, D), jnp.float32),
            pltpu.MemorySpace.VMEM_SHARED((NR, D), jnp.float32),
            pltpu.SemaphoreType.DMA,
            pltpu.SemaphoreType.DMA,
            pltpu.SemaphoreType.DMA,
            pltpu.SemaphoreType.DMA,
            pltpu.SemaphoreType.DMA,
        ],
    )
    return f(m, src3, dst3, zrows)


def _stage_a_body(no_ref, h_ref, w_ref, o_ref):
    o_ref[...] = jnp.dot(h_ref[...] * no_ref[...], w_ref[...],
                         preferred_element_type=jnp.float32)


def _stage_a(norm_out, h, W1):
    return pl.pallas_call(
        _stage_a_body,
        out_shape=jax.ShapeDtypeStruct((NR, D), jnp.float32),
        grid=(N // RB,),
        in_specs=[
            pl.BlockSpec((RB, 1), lambda i: (i, 0)),
            pl.BlockSpec((RB, D), lambda i: (i, 0)),
            pl.BlockSpec((D, D), lambda i: (0, 0)),
        ],
        out_specs=pl.BlockSpec((RB, D), lambda i: (i, 0)),
    )(norm_out, h, W1)


def _stage_b_body(p_ref, ni_ref, b1_ref, g_ref, be_ref, no_ref, w2_ref,
                  m2_ref, x_vmem, s_vmem):
    ph = pl.program_id(0)
    i = pl.program_id(1)

    @pl.when(ph == 0)
    def _():
        x = (p_ref[0] + p_ref[1]) * ni_ref[...] + b1_ref[...]
        x_vmem[i] = x
        s0 = jnp.sum(x, axis=0)[None, :]
        s1 = jnp.sum(x * x, axis=0)[None, :]

        @pl.when(i == 0)
        def _():
            s_vmem[0:1, :] = s0
            s_vmem[1:2, :] = s1

        @pl.when(i > 0)
        def _():
            s_vmem[0:1, :] += s0
            s_vmem[1:2, :] += s1

    @pl.when(ph == 1)
    def _():
        inv_n = jnp.float32(1.0 / N)
        mean = s_vmem[0:1, :] * inv_n
        ex2 = s_vmem[1:2, :] * inv_n
        var = ex2 - mean * mean
        xb = x_vmem[i]
        xn = (xb - mean) * lax.rsqrt(var + 1e-5) * g_ref[...] + be_ref[...]
        r = jnp.maximum(xn, 0.0) * no_ref[...]
        m2_ref[...] = jnp.dot(r, w2_ref[...],
                              preferred_element_type=jnp.float32)


def _stage_b(p1, norm_in, b1, gamma, beta, norm_out, W2):
    return pl.pallas_call(
        _stage_b_body,
        out_shape=jax.ShapeDtypeStruct((NR, D), jnp.float32),
        grid=(2, N // RB),
        in_specs=[
            pl.BlockSpec((2, RB, D), lambda p, i: (0, i * (1 - p), 0)),
            pl.BlockSpec((RB, 1), lambda p, i: (i, 0)),
            pl.BlockSpec((1, D), lambda p, i: (0, 0)),
            pl.BlockSpec((1, D), lambda p, i: (0, 0)),
            pl.BlockSpec((1, D), lambda p, i: (0, 0)),
            pl.BlockSpec((RB, 1), lambda p, i: (i, 0)),
            pl.BlockSpec((D, D), lambda p, i: (0, 0)),
        ],
        out_specs=pl.BlockSpec((RB, D), lambda p, i: (i * p, 0)),
        scratch_shapes=[
            pltpu.VMEM((N // RB, RB, D), jnp.float32),
            pltpu.VMEM((8, D), jnp.float32),
        ],
    )(p1, norm_in, b1, gamma, beta, norm_out, W2)


def _stage_c_body(q_ref, ni_ref, b2_ref, o_ref):
    o_ref[...] = (q_ref[0] + q_ref[1]) * ni_ref[...] + b2_ref[...]


def _stage_c(p2, norm_in, b2):
    return pl.pallas_call(
        _stage_c_body,
        out_shape=jax.ShapeDtypeStruct((N, D), jnp.float32),
        grid=(N // RB,),
        in_specs=[
            pl.BlockSpec((2, RB, D), lambda i: (0, i, 0)),
            pl.BlockSpec((RB, 1), lambda i: (i, 0)),
            pl.BlockSpec((1, D), lambda i: (0, 0)),
        ],
        out_specs=pl.BlockSpec((RB, D), lambda i: (i, 0)),
    )(p2, norm_in, b2)


def kernel(h, edge_index, W1, b1, gamma, beta, W2, b2):
    npad = EPTP - EPT
    padi = (N + jnp.arange(npad, dtype=jnp.int32) % (NR - N))[None, :]
    src2 = edge_index[0].reshape(NW, EPT)
    dst2 = edge_index[1].reshape(NW, EPT)
    pads = jnp.broadcast_to(padi, (NW, npad))
    src3 = jnp.concatenate([src2, pads], axis=1).reshape(NW, NWIN, C)
    dst3 = jnp.concatenate([dst2, pads], axis=1).reshape(NW, NWIN, C)

    zs2 = jnp.zeros((2, NR), jnp.float32)
    zrows = jnp.zeros((NS, RPT, D), jnp.float32)

    degp = _deg_call(src3, dst3, zs2)
    deg_out = degp[0, 0, :N] + degp[1, 0, :N]
    deg_in = degp[0, 1, :N] + degp[1, 1, :N]
    norm_out = jnp.where(deg_out > 0,
                         lax.rsqrt(jnp.maximum(deg_out, 1.0)),
                         0.0).reshape(N, 1)
    norm_in = jnp.where(deg_in > 0,
                        lax.rsqrt(jnp.maximum(deg_in, 1.0)),
                        0.0).reshape(N, 1)

    m1 = _stage_a(norm_out, h, W1)
    p1 = _agg_call(m1, src3, dst3, zrows)
    m2 = _stage_b(p1, norm_in, b1.reshape(1, D), gamma.reshape(1, D),
                  beta.reshape(1, D), norm_out, W2)
    p2 = _agg_call(m2, src3, dst3, zrows)
    return _stage_c(p2, norm_in, b2.reshape(1, D))


# speedup vs baseline: 1.0839x; 1.0013x over previous
import jax
import jax.numpy as jnp
from jax import lax
from jax.experimental import pallas as pl
from jax.experimental.pallas import tpu as pltpu
import jax.experimental.pallas.tpu_sc as plsc

N = 10000
D = 128
E = 320000
NC = 2
NS = 16
NW = NC * NS
EPT = E // NW
C = 128
NWIN = 80
EPTP = NWIN * C
NR = 10240
RPT = NR // NS
RB = 1000


def _mesh():
    return plsc.VectorSubcoreMesh(core_axis_name="c", subcore_axis_name="s")


def _deg_body(src3, dst3, zs2, degp, idx_s, idx_d, ones_v, acc_o, acc_i,
              sem_a, sem_b):
    cid = lax.axis_index("c")
    sid = lax.axis_index("s")
    wid = cid * NS + sid

    for j in range(8):
        ones_v[pl.ds(j * 16, 16)] = jnp.ones((16,), jnp.float32)

    @pl.when(sid == 0)
    def _():
        pltpu.sync_copy(zs2.at[0], acc_o)
        pltpu.sync_copy(zs2.at[1], acc_i)

    pltpu.sync_copy(src3.at[wid], idx_s)
    pltpu.sync_copy(dst3.at[wid], idx_d)
    plsc.subcore_barrier()

    def body(j4, carry):
        ds = []
        for t in range(4):
            j = j4 * 4 + t
            ds.append(pltpu.async_copy(ones_v, acc_o.at[idx_s.at[j]],
                                       sem_a, add=True))
            ds.append(pltpu.async_copy(ones_v, acc_i.at[idx_d.at[j]],
                                       sem_b, add=True))
        for dsc in ds:
            dsc.wait()
        return carry

    lax.fori_loop(0, NWIN // 4, body, 0)
    plsc.subcore_barrier()

    off = sid * RPT
    pltpu.sync_copy(acc_o.at[pl.ds(off, RPT)], degp.at[cid, 0, pl.ds(off, RPT)])
    pltpu.sync_copy(acc_i.at[pl.ds(off, RPT)], degp.at[cid, 1, pl.ds(off, RPT)])


def _deg_call(src3, dst3, zs2):
    f = pl.kernel(
        _deg_body,
        out_type=jax.ShapeDtypeStruct((NC, 2, NR), jnp.float32),
        mesh=_mesh(),
        scratch_types=[
            pltpu.VMEM((NWIN, C), jnp.int32),
            pltpu.VMEM((NWIN, C), jnp.int32),
            pltpu.VMEM((C,), jnp.float32),
            pltpu.MemorySpace.VMEM_SHARED((NR,), jnp.float32),
            pltpu.MemorySpace.VMEM_SHARED((NR,), jnp.float32),
            pltpu.SemaphoreType.DMA,
            pltpu.SemaphoreType.DMA,
        ],
    )
    return f(src3, dst3, zs2)


ICH = 8
NCHK = NWIN // ICH


def _agg_body(m_hbm, src3, dst3, zrows, part, ics, icd, rows, acc,
              sem_g0, sem_g1, sem_s0, sem_s1, sem_i):
    cid = lax.axis_index("c")
    sid = lax.axis_index("s")
    wid = cid * NS + sid

    pltpu.sync_copy(zrows.at[sid], acc.at[pl.ds(sid * RPT, RPT)])

    sem_s = (sem_s0, sem_s1)
    sem_g = (sem_g0, sem_g1)

    Q = C // 4

    def _gather(slot, k, b):
        return [pltpu.async_copy(m_hbm.at[ics.at[slot, k, pl.ds(q * Q, Q)]],
                                 rows.at[b, pl.ds(q * Q, Q)], sem_g[b])
                for q in range(4)]

    pltpu.async_copy(src3.at[wid, pl.ds(0, ICH)], ics.at[0], sem_i)
    pltpu.async_copy(dst3.at[wid, pl.ds(0, ICH)], icd.at[0], sem_i)
    plsc.subcore_barrier()

    def chunk(cix, carry):
        slot = lax.rem(cix, 2)
        nslot = 1 - slot
        pltpu.make_async_copy(src3.at[wid, pl.ds(0, ICH)], ics.at[slot],
                              sem_i).wait()
        pltpu.make_async_copy(dst3.at[wid, pl.ds(0, ICH)], icd.at[slot],
                              sem_i).wait()

        @pl.when(cix > 0)
        def _():
            pltpu.make_async_copy(rows.at[0], acc.at[icd.at[0, 0]],
                                  sem_s0).wait()
            pltpu.make_async_copy(rows.at[1], acc.at[icd.at[1, 0]],
                                  sem_s1).wait()

        @pl.when(cix + 1 < NCHK)
        def _():
            nbase = (cix + 1) * ICH
            pltpu.async_copy(src3.at[wid, pl.ds(nbase, ICH)], ics.at[nslot],
                             sem_i)
            pltpu.async_copy(dst3.at[wid, pl.ds(nbase, ICH)], icd.at[nslot],
                             sem_i)

        g = [_gather(slot, 0, 0), None]
        s = [None, None]
        for k in range(ICH):
            b = k % 2
            nb = 1 - b
            if k + 1 < ICH:
                if s[nb] is not None:
                    s[nb].wait()
                g[nb] = _gather(slot, k + 1, nb)
            for gd in g[b]:
                gd.wait()
            s[b] = pltpu.async_copy(rows.at[b], acc.at[icd.at[slot, k]],
                                    sem_s[b], add=True)
        return carry

    lax.fori_loop(0, NCHK, chunk, 0)
    pltpu.make_async_copy(rows.at[0], acc.at[icd.at[0, 0]], sem_s0).wait()
    pltpu.make_async_copy(rows.at[1], acc.at[icd.at[1, 0]], sem_s1).wait()
    plsc.subcore_barrier()

    pltpu.sync_copy(acc.at[pl.ds(sid * RPT, RPT)],
                    part.at[cid, pl.ds(sid * RPT, RPT)])


def _agg_call(m, src3, dst3, zrows):
    f = pl.kernel(
        _agg_body,
        out_type=jax.ShapeDtypeStruct((NC, NR, D), jnp.float32),
        mesh=_mesh(),
        compiler_params=pltpu.CompilerParams(use_tc_tiling_on_sc=True),
        scratch_types=[
            pltpu.VMEM((2, ICH, C), jnp.int32),
            pltpu.VMEM((2, ICH, C), jnp.int32),
            pltpu.VMEM((2, C, D), jnp.float32),
            pltpu.MemorySpace.VMEM_SHARED((NR, D), jnp.float32),
            pltpu.SemaphoreType.DMA,
            pltpu.SemaphoreType.DMA,
            pltpu.SemaphoreType.DMA,
            pltpu.SemaphoreType.DMA,
            pltpu.SemaphoreType.DMA,
        ],
    )
    return f(m, src3, dst3, zrows)


def _stage_a_body(no_ref, h_ref, w_ref, o_ref):
    o_ref[...] = jnp.dot(h_ref[...] * no_ref[...], w_ref[...],
                         preferred_element_type=jnp.float32)


def _stage_a(norm_out, h, W1):
    return pl.pallas_call(
        _stage_a_body,
        out_shape=jax.ShapeDtypeStruct((NR, D), jnp.float32),
        grid=(N // RB,),
        in_specs=[
            pl.BlockSpec((RB, 1), lambda i: (i, 0)),
            pl.BlockSpec((RB, D), lambda i: (i, 0)),
            pl.BlockSpec((D, D), lambda i: (0, 0)),
        ],
        out_specs=pl.BlockSpec((RB, D), lambda i: (i, 0)),
    )(norm_out, h, W1)


def _stage_b_body(p_ref, ni_ref, b1_ref, g_ref, be_ref, no_ref, w2_ref,
                  m2_ref, x_vmem, s_vmem):
    ph = pl.program_id(0)
    i = pl.program_id(1)

    @pl.when(ph == 0)
    def _():
        x = (p_ref[0] + p_ref[1]) * ni_ref[...] + b1_ref[...]
        x_vmem[i] = x
        s0 = jnp.sum(x, axis=0)[None, :]
        s1 = jnp.sum(x * x, axis=0)[None, :]

        @pl.when(i == 0)
        def _():
            s_vmem[0:1, :] = s0
            s_vmem[1:2, :] = s1

        @pl.when(i > 0)
        def _():
            s_vmem[0:1, :] += s0
            s_vmem[1:2, :] += s1

    @pl.when(ph == 1)
    def _():
        inv_n = jnp.float32(1.0 / N)
        mean = s_vmem[0:1, :] * inv_n
        ex2 = s_vmem[1:2, :] * inv_n
        var = ex2 - mean * mean
        xb = x_vmem[i]
        xn = (xb - mean) * lax.rsqrt(var + 1e-5) * g_ref[...] + be_ref[...]
        r = jnp.maximum(xn, 0.0) * no_ref[...]
        m2_ref[...] = jnp.dot(r, w2_ref[...],
                              preferred_element_type=jnp.float32)


def _stage_b(p1, norm_in, b1, gamma, beta, norm_out, W2):
    return pl.pallas_call(
        _stage_b_body,
        out_shape=jax.ShapeDtypeStruct((NR, D), jnp.float32),
        grid=(2, N // RB),
        in_specs=[
            pl.BlockSpec((2, RB, D), lambda p, i: (0, i * (1 - p), 0)),
            pl.BlockSpec((RB, 1), lambda p, i: (i, 0)),
            pl.BlockSpec((1, D), lambda p, i: (0, 0)),
            pl.BlockSpec((1, D), lambda p, i: (0, 0)),
            pl.BlockSpec((1, D), lambda p, i: (0, 0)),
            pl.BlockSpec((RB, 1), lambda p, i: (i, 0)),
            pl.BlockSpec((D, D), lambda p, i: (0, 0)),
        ],
        out_specs=pl.BlockSpec((RB, D), lambda p, i: (i * p, 0)),
        scratch_shapes=[
            pltpu.VMEM((N // RB, RB, D), jnp.float32),
            pltpu.VMEM((8, D), jnp.float32),
        ],
    )(p1, norm_in, b1, gamma, beta, norm_out, W2)


def _stage_c_body(q_ref, ni_ref, b2_ref, o_ref):
    o_ref[...] = (q_ref[0] + q_ref[1]) * ni_ref[...] + b2_ref[...]


def _stage_c(p2, norm_in, b2):
    return pl.pallas_call(
        _stage_c_body,
        out_shape=jax.ShapeDtypeStruct((N, D), jnp.float32),
        grid=(N // RB,),
        in_specs=[
            pl.BlockSpec((2, RB, D), lambda i: (0, i, 0)),
            pl.BlockSpec((RB, 1), lambda i: (i, 0)),
            pl.BlockSpec((1, D), lambda i: (0, 0)),
        ],
        out_specs=pl.BlockSpec((RB, D), lambda i: (i, 0)),
    )(p2, norm_in, b2)


def kernel(h, edge_index, W1, b1, gamma, beta, W2, b2):
    npad = EPTP - EPT
    padi = (N + jnp.arange(npad, dtype=jnp.int32) % (NR - N))[None, :]
    src2 = edge_index[0].reshape(NW, EPT)
    dst2 = edge_index[1].reshape(NW, EPT)
    pads = jnp.broadcast_to(padi, (NW, npad))
    src3 = jnp.concatenate([src2, pads], axis=1).reshape(NW, NWIN, C)
    dst3 = jnp.concatenate([dst2, pads], axis=1).reshape(NW, NWIN, C)

    zs2 = jnp.zeros((2, NR), jnp.float32)
    zrows = jnp.zeros((NS, RPT, D), jnp.float32)

    degp = _deg_call(src3, dst3, zs2)
    deg_out = degp[0, 0, :N] + degp[1, 0, :N]
    deg_in = degp[0, 1, :N] + degp[1, 1, :N]
    norm_out = jnp.where(deg_out > 0,
                         lax.rsqrt(jnp.maximum(deg_out, 1.0)),
                         0.0).reshape(N, 1)
    norm_in = jnp.where(deg_in > 0,
                        lax.rsqrt(jnp.maximum(deg_in, 1.0)),
                        0.0).reshape(N, 1)

    m1 = _stage_a(norm_out, h, W1)
    p1 = _agg_call(m1, src3, dst3, zrows)
    m2 = _stage_b(p1, norm_in, b1.reshape(1, D), gamma.reshape(1, D),
                  beta.reshape(1, D), norm_out, W2)
    p2 = _agg_call(m2, src3, dst3, zrows)
    return _stage_c(p2, norm_in, b2.reshape(1, D))
